# baseline (speedup 1.0000x reference)
import functools

import jax
import jax.numpy as jnp
from jax import lax
from jax.experimental import pallas as pl
from jax.experimental.pallas import tpu as pltpu
from jax.experimental.pallas import tpu_sc as plsc

NW = 32


def _sc_kernel_body(x_hbm, o_hbm, xv, av, accr, sr, *, k, nrows, d):
    nv = d // 16
    wid = lax.axis_index("s") * 2 + lax.axis_index("c")
    rows_pw = nrows // NW

    def row_body(ri, _):
        row = wid * rows_pw + ri
        pltpu.sync_copy(x_hbm.at[row], xv)

        def silu_body(vi, _):
            v = xv[pl.ds(vi * 16, 16)]
            s = v / (1.0 + jnp.exp(-v))
            av[pl.ds(vi * 16, 16)] = s
            return 0

        lax.fori_loop(0, nv, silu_body, 0, unroll=False)

        zero = jnp.zeros((16,), jnp.int32)
        tv = zero
        for bit in range(30, 8, -1):
            candv = tv | jnp.int32(1 << bit)
            accr[pl.ds(0, 16)] = zero

            def cnt_body(vi, _, candv=candv):
                bits = jax.lax.bitcast_convert_type(
                    av[pl.ds(vi * 16, 16)], jnp.int32
                ) & jnp.int32(0x7FFFFFFF)
                one = jnp.full((16,), 1, jnp.int32)
                accr[pl.ds(0, 16)] = accr[pl.ds(0, 16)] + jnp.where(
                    bits >= candv, one, zero
                )
                return 0

            lax.fori_loop(0, nv, cnt_body, 0, unroll=False)
            cntv = accr[pl.ds(0, 16)]
            iota = lax.iota(jnp.int32, 16)
            dnums = lax.GatherDimensionNumbers(
                offset_dims=(),
                collapsed_slice_dims=(0,),
                start_index_map=(0,),
            )
            for sh in (1, 2, 4, 8):
                perm = (iota ^ sh)[:, None]
                cntv = cntv + lax.gather(
                    cntv,
                    perm,
                    dnums,
                    (1,),
                    mode=lax.GatherScatterMode.PROMISE_IN_BOUNDS,
                )
            tv = jnp.where(cntv >= k, candv, tv)

        def mask_body(vi, _):
            sl = pl.ds(vi * 16, 16)
            s = av[sl]
            bits = jax.lax.bitcast_convert_type(s, jnp.int32) & jnp.int32(
                0x7FFFFFFF
            )
            xv[sl] = jnp.where(bits >= tv, s, 0.0)
            return 0

        lax.fori_loop(0, nv, mask_body, 0, unroll=False)
        pltpu.sync_copy(xv, o_hbm.at[row])
        return 0

    lax.fori_loop(0, rows_pw, row_body, 0, unroll=False)


def kernel(x):
    b, s, d = x.shape
    k = max(1, int(d * 0.5))
    rows = b * s
    xr = x.reshape(rows, d)
    mesh = plsc.VectorSubcoreMesh(core_axis_name="c", subcore_axis_name="s")
    sc_call = functools.partial(
        pl.kernel,
        mesh=mesh,
        out_type=jax.ShapeDtypeStruct((rows, d), jnp.float32),
        scratch_types=[
            pltpu.VMEM((d,), jnp.float32),
            pltpu.VMEM((d,), jnp.float32),
            pltpu.VMEM((16,), jnp.int32),
            pltpu.SMEM((1,), jnp.int32),
        ],
    )(functools.partial(_sc_kernel_body, k=k, nrows=rows, d=d))
    out = sc_call(xr)
    return out.reshape(b, s, d)


# speedup vs baseline: 25.4045x; 25.4045x over previous
import functools

import jax
import jax.numpy as jnp
from jax.experimental import pallas as pl

ROWS_PER_BLOCK = 256
GROUPS = 4


def _chunks(d):
    return [slice(c * 128, (c + 1) * 128) for c in range(d // 128)]


def _count_ge16(hi_g, c16, d):
    sls = _chunks(d)
    acc = (hi_g[:, sls[0]] >= c16).astype(jnp.int16)
    for sl in sls[1:]:
        acc = acc + (hi_g[:, sl] >= c16).astype(jnp.int16)
    return jnp.sum(acc.astype(jnp.int32), axis=1, keepdims=True)


def _count_lo16(lo_g, eq_g, c16, d):
    sls = _chunks(d)
    z = jnp.zeros((lo_g.shape[0], 128), jnp.int16)
    acc = jnp.where(lo_g[:, sls[0]] >= c16, eq_g[:, sls[0]], z)
    for sl in sls[1:]:
        acc = acc + jnp.where(lo_g[:, sl] >= c16, eq_g[:, sl], z)
    return jnp.sum(acc.astype(jnp.int32), axis=1, keepdims=True)


def _topk_mask_kernel(x_ref, o_ref, *, k):
    x = x_ref[...]
    a = x * jax.nn.sigmoid(x)
    bits = jax.lax.bitcast_convert_type(a, jnp.int32) & jnp.int32(0x7FFFFFFF)
    r = x.shape[0]
    d = x.shape[1]
    h = r // GROUPS
    sls = _chunks(d)
    rows = [slice(g * h, (g + 1) * h) for g in range(GROUPS)]

    hi = (bits >> 16).astype(jnp.int16)
    hig = [hi[rs] for rs in rows]
    bitsg = [bits[rs] for rs in rows]

    t0 = jnp.zeros((h, 128), jnp.int32)

    def body16(i, ts):
        bit = jnp.int32(1) << (14 - i)
        out = []
        for g in range(GROUPS):
            cand = ts[g] | bit
            cnt = _count_ge16(hig[g], cand.astype(jnp.int16), d)
            out.append(
                jnp.where(jnp.broadcast_to(cnt, (h, 128)) >= k, cand, ts[g])
            )
        return tuple(out)

    t16 = jax.lax.fori_loop(0, 15, body16, (t0,) * GROUPS, unroll=False)

    lo = (bits ^ jnp.int32(0x8000)).astype(jnp.int16)
    log = [lo[rs] for rs in rows]
    eqg = []
    k2g = []
    for g in range(GROUPS):
        p16 = t16[g].astype(jnp.int16)
        eqg.append(
            jnp.concatenate(
                [(hig[g][:, sl] == p16).astype(jnp.int16) for sl in sls], axis=1
            )
        )
        cnt_gt = _count_ge16(hig[g], (t16[g] + 1).astype(jnp.int16), d)
        k2g.append(jnp.broadcast_to(k - cnt_gt, (h, 128)))

    def body_lo(i, ts):
        bit = jnp.int32(1) << (15 - i)
        out = []
        for g in range(GROUPS):
            cand = ts[g] | bit
            c16 = ((cand ^ jnp.int32(0x8000)) << 16 >> 16).astype(jnp.int16)
            cnt = _count_lo16(log[g], eqg[g], c16, d)
            out.append(
                jnp.where(jnp.broadcast_to(cnt, (h, 128)) >= k2g[g], cand, ts[g])
            )
        return tuple(out)

    tf = jax.lax.fori_loop(
        0, 7, body_lo, tuple(t << 16 for t in t16), unroll=False
    )

    for g in range(GROUPS):
        for sl in sls:
            o_ref[rows[g], sl] = jnp.where(
                bitsg[g][:, sl] >= tf[g], a[rows[g], sl], 0.0
            )


def kernel(x):
    b, s, d = x.shape
    k = max(1, int(d * 0.5))
    xr = x.reshape(b * s, d)
    rows = b * s
    out = pl.pallas_call(
        functools.partial(_topk_mask_kernel, k=k),
        grid=(rows // ROWS_PER_BLOCK,),
        in_specs=[pl.BlockSpec((ROWS_PER_BLOCK, d), lambda i: (i, 0))],
        out_specs=pl.BlockSpec((ROWS_PER_BLOCK, d), lambda i: (i, 0)),
        out_shape=jax.ShapeDtypeStruct((rows, d), jnp.float32),
    )(xr)
    return out.reshape(b, s, d)
